# baseline (speedup 1.0000x reference)
import functools

import jax
import jax.numpy as jnp
from jax.experimental import pallas as pl
from jax.experimental.pallas import tpu as pltpu


M_TILE = 2048
M_SUB = 256
W1_CHUNKS = 4


def _mlp_kernel(x_ref, w1_ref, b1_ref, w2_ref, b2_ref, out_ref, w1b_ref):
    i = pl.program_id(0)
    d = w1b_ref.shape[0]
    kc = d // W1_CHUNKS

    @pl.when(i < W1_CHUNKS)
    def _():
        w1b_ref[pl.ds(i * kc, kc), :] = w1_ref[...].astype(jnp.bfloat16)

    @pl.when(i >= W1_CHUNKS)
    def _():
        for s in range(M_TILE // M_SUB):
            x = x_ref[pl.ds(s * M_SUB, M_SUB), :].astype(jnp.bfloat16)
            h = jnp.dot(x, w1b_ref[...], preferred_element_type=jnp.float32)
            h = jnp.maximum(h + b1_ref[...], 0.0)
            logits = jnp.sum(h * w2_ref[...], axis=1) + b2_ref[0, 0]
            out_ref[0, 0, pl.ds(s * M_SUB, M_SUB)] = (
                0.5 * jnp.tanh(0.5 * logits) + 0.5)


@functools.partial(jax.jit, static_argnames=())
def kernel(img_features, W1, b1, W2, b2):
    n, d = img_features.shape
    num_tiles = n // M_TILE
    kc = d // W1_CHUNKS
    b1r = b1.reshape(1, d)
    w2r = W2.reshape(1, d)
    b2r = b2.reshape(1, 1)
    out = pl.pallas_call(
        _mlp_kernel,
        grid=(num_tiles + W1_CHUNKS,),
        in_specs=[
            pl.BlockSpec((M_TILE, d),
                         lambda i: (jnp.maximum(i - W1_CHUNKS, 0), 0)),
            pl.BlockSpec((kc, d), lambda i: (jnp.minimum(i, W1_CHUNKS - 1), 0)),
            pl.BlockSpec((1, d), lambda i: (0, 0)),
            pl.BlockSpec((1, d), lambda i: (0, 0)),
            pl.BlockSpec((1, 1), lambda i: (0, 0)),
        ],
        out_specs=pl.BlockSpec(
            (1, 1, M_TILE), lambda i: (jnp.maximum(i - W1_CHUNKS, 0), 0, 0)),
        out_shape=jax.ShapeDtypeStruct((num_tiles, 1, M_TILE), jnp.float32),
        scratch_shapes=[pltpu.VMEM((d, d), jnp.bfloat16)],
    )(img_features, W1, b1r, w2r, b2r)
    return out.reshape(n, 1)


# speedup vs baseline: 1.0312x; 1.0312x over previous
import functools

import jax
import jax.numpy as jnp
from jax.experimental import pallas as pl
from jax.experimental.pallas import tpu as pltpu


M_TILE = 2048
M_SUB = 512
W1_CHUNKS = 4


def _mlp_kernel(x_ref, w1_ref, b1_ref, w2_ref, b2_ref, out_ref, w1b_ref):
    i = pl.program_id(0)
    d = w1b_ref.shape[0]
    kc = d // W1_CHUNKS

    @pl.when(i < W1_CHUNKS)
    def _():
        w1b_ref[pl.ds(i * kc, kc), :] = w1_ref[...].astype(jnp.bfloat16)

    @pl.when(i >= W1_CHUNKS)
    def _():
        for s in range(M_TILE // M_SUB):
            x = x_ref[pl.ds(s * M_SUB, M_SUB), :].astype(jnp.bfloat16)
            h = jnp.dot(x, w1b_ref[...], preferred_element_type=jnp.float32)
            h = jnp.maximum(h + b1_ref[...], 0.0)
            logits = jnp.sum(h * w2_ref[...], axis=1) + b2_ref[0, 0]
            out_ref[0, 0, pl.ds(s * M_SUB, M_SUB)] = (
                0.5 * jnp.tanh(0.5 * logits) + 0.5)


@functools.partial(jax.jit, static_argnames=())
def kernel(img_features, W1, b1, W2, b2):
    n, d = img_features.shape
    num_tiles = n // M_TILE
    kc = d // W1_CHUNKS
    b1r = b1.reshape(1, d)
    w2r = W2.reshape(1, d)
    b2r = b2.reshape(1, 1)
    out = pl.pallas_call(
        _mlp_kernel,
        grid=(num_tiles + W1_CHUNKS,),
        in_specs=[
            pl.BlockSpec((M_TILE, d),
                         lambda i: (jnp.maximum(i - W1_CHUNKS, 0), 0)),
            pl.BlockSpec((kc, d), lambda i: (jnp.minimum(i, W1_CHUNKS - 1), 0)),
            pl.BlockSpec((1, d), lambda i: (0, 0)),
            pl.BlockSpec((1, d), lambda i: (0, 0)),
            pl.BlockSpec((1, 1), lambda i: (0, 0)),
        ],
        out_specs=pl.BlockSpec(
            (1, 1, M_TILE), lambda i: (jnp.maximum(i - W1_CHUNKS, 0), 0, 0)),
        out_shape=jax.ShapeDtypeStruct((num_tiles, 1, M_TILE), jnp.float32),
        scratch_shapes=[pltpu.VMEM((d, d), jnp.bfloat16)],
    )(img_features, W1, b1r, w2r, b2r)
    return out.reshape(n, 1)


# speedup vs baseline: 1.0326x; 1.0013x over previous
import functools

import jax
import jax.numpy as jnp
from jax.experimental import pallas as pl
from jax.experimental.pallas import tpu as pltpu


M_TILE = 2048
M_SUB = 512
W1_CHUNKS = 4


def _mlp_kernel(x_ref, w1_ref, b1_ref, w2_ref, b2_ref, out_ref, w1b_ref,
                xb_ref):
    i = pl.program_id(0)
    d = w1b_ref.shape[0]
    kc = d // W1_CHUNKS

    @pl.when(i < W1_CHUNKS)
    def _():
        w1b_ref[pl.ds(i * kc, kc), :] = w1_ref[...].astype(jnp.bfloat16)

    @pl.when(i >= W1_CHUNKS)
    def _():
        xb_ref[...] = x_ref[...].astype(jnp.bfloat16)
        for s in range(M_TILE // M_SUB):
            x = xb_ref[pl.ds(s * M_SUB, M_SUB), :]
            h = jnp.dot(x, w1b_ref[...], preferred_element_type=jnp.float32)
            h = jnp.maximum(h + b1_ref[...], 0.0)
            logits = jnp.sum(h * w2_ref[...], axis=1) + b2_ref[0, 0]
            out_ref[0, 0, pl.ds(s * M_SUB, M_SUB)] = (
                0.5 * jnp.tanh(0.5 * logits) + 0.5)


@functools.partial(jax.jit, static_argnames=())
def kernel(img_features, W1, b1, W2, b2):
    n, d = img_features.shape
    num_tiles = n // M_TILE
    kc = d // W1_CHUNKS
    b1r = b1.reshape(1, d)
    w2r = W2.reshape(1, d)
    b2r = b2.reshape(1, 1)
    out = pl.pallas_call(
        _mlp_kernel,
        grid=(num_tiles + W1_CHUNKS,),
        in_specs=[
            pl.BlockSpec((M_TILE, d),
                         lambda i: (jnp.maximum(i - W1_CHUNKS, 0), 0)),
            pl.BlockSpec((kc, d), lambda i: (jnp.minimum(i, W1_CHUNKS - 1), 0)),
            pl.BlockSpec((1, d), lambda i: (0, 0)),
            pl.BlockSpec((1, d), lambda i: (0, 0)),
            pl.BlockSpec((1, 1), lambda i: (0, 0)),
        ],
        out_specs=pl.BlockSpec(
            (1, 1, M_TILE), lambda i: (jnp.maximum(i - W1_CHUNKS, 0), 0, 0)),
        out_shape=jax.ShapeDtypeStruct((num_tiles, 1, M_TILE), jnp.float32),
        scratch_shapes=[pltpu.VMEM((d, d), jnp.bfloat16),
                        pltpu.VMEM((M_TILE, d), jnp.bfloat16)],
    )(img_features, W1, b1r, w2r, b2r)
    return out.reshape(n, 1)
